# baseline (speedup 1.0000x reference)
import jax
import jax.numpy as jnp
from jax.experimental import pallas as pl

_SEQ_BLK = 512


def _add_kernel(x_ref, pos_ref, o_ref):
    o_ref[...] = x_ref[...] + pos_ref[...][None, :, :]


def kernel(x, pos_table):
    batch, seq_len, d_model = x.shape
    pos = pos_table[:seq_len]
    n_s = seq_len // _SEQ_BLK
    return pl.pallas_call(
        _add_kernel,
        grid=(n_s,),
        in_specs=[
            pl.BlockSpec((batch, _SEQ_BLK, d_model), lambda s: (0, s, 0)),
            pl.BlockSpec((_SEQ_BLK, d_model), lambda s: (s, 0)),
        ],
        out_specs=pl.BlockSpec((batch, _SEQ_BLK, d_model), lambda s: (0, s, 0)),
        out_shape=jax.ShapeDtypeStruct((batch, seq_len, d_model), x.dtype),
    )(x, pos)
